# baseline (speedup 1.0000x reference)
import functools

import jax
import jax.numpy as jnp
from jax import lax
from jax.experimental import pallas as pl
from jax.experimental.pallas import tpu as pltpu
from jax.experimental.pallas import tpu_sc as plsc

D_MODEL = 64
SCALE = 8.0

NUM_CORES = 2
NUM_SUBCORES = 16
NUM_WORKERS = NUM_CORES * NUM_SUBCORES

CHUNK = 512


@functools.partial(jax.jit, static_argnums=(2,))
def _embed(idx_flat, table, total):
    b_per_w = total // NUM_WORKERS
    n_chunks = b_per_w // CHUNK

    mesh = plsc.VectorSubcoreMesh(core_axis_name="c", subcore_axis_name="s")

    @functools.partial(
        pl.kernel,
        mesh=mesh,
        out_type=jax.ShapeDtypeStruct((total, D_MODEL), jnp.float32),
        scratch_types=[
            pltpu.VMEM((CHUNK,), jnp.int32),
            pltpu.VMEM((CHUNK, D_MODEL), jnp.float32),
            pltpu.SemaphoreType.DMA,
        ],
        compiler_params=pltpu.CompilerParams(use_tc_tiling_on_sc=False),
    )
    def emb(idx_hbm, table_hbm, out_hbm, idx_v, rows_v, sem):
        wid = lax.axis_index("s") * NUM_CORES + lax.axis_index("c")
        base = wid * b_per_w

        def chunk_body(g, carry):
            off = base + g * CHUNK
            pltpu.sync_copy(idx_hbm.at[pl.ds(off, CHUNK)], idx_v)
            pltpu.async_copy(table_hbm.at[idx_v], rows_v, sem).wait()

            def row_body(r, c):
                for k in range(D_MODEL // 16):
                    sl = pl.ds(k * 16, 16)
                    rows_v[r, sl] = rows_v[r, sl] * SCALE
                return c

            lax.fori_loop(0, CHUNK, row_body, 0)
            pltpu.sync_copy(rows_v, out_hbm.at[pl.ds(off, CHUNK)])
            return carry

        lax.fori_loop(0, n_chunks, chunk_body, 0)

    return emb(idx_flat, table)


def kernel(x, table):
    rows, cols = x.shape
    total = rows * cols
    idx_flat = x.reshape(total).astype(jnp.int32)
    out = _embed(idx_flat, table, total)
    return out.reshape(rows, cols, D_MODEL)


# speedup vs baseline: 1.1322x; 1.1322x over previous
import functools

import jax
import jax.numpy as jnp
from jax import lax
from jax.experimental import pallas as pl
from jax.experimental.pallas import tpu as pltpu
from jax.experimental.pallas import tpu_sc as plsc

D_MODEL = 64
SCALE = 8.0

NUM_CORES = 2
NUM_SUBCORES = 16
NUM_WORKERS = NUM_CORES * NUM_SUBCORES

CHUNK = 800
NBUF = 2


@functools.partial(jax.jit, static_argnums=(2,))
def _embed(idx_flat, table, total):
    b_per_w = total // NUM_WORKERS
    n_chunks = b_per_w // CHUNK
    n_outer = n_chunks // NBUF

    mesh = plsc.VectorSubcoreMesh(core_axis_name="c", subcore_axis_name="s")

    @functools.partial(
        pl.kernel,
        mesh=mesh,
        out_type=jax.ShapeDtypeStruct((total, D_MODEL), jnp.float32),
        scratch_types=[
            pltpu.VMEM((b_per_w,), jnp.int32),
            *[pltpu.VMEM((CHUNK, D_MODEL), jnp.float32) for _ in range(NBUF)],
            *[pltpu.SemaphoreType.DMA for _ in range(2 * NBUF)],
        ],
        compiler_params=pltpu.CompilerParams(use_tc_tiling_on_sc=False),
    )
    def emb(idx_hbm, table_hbm, out_hbm, idx_v, rows0, rows1, g0, g1, o0, o1):
        rows = (rows0, rows1)
        gsem = (g0, g1)
        osem = (o0, o1)
        wid = lax.axis_index("s") * NUM_CORES + lax.axis_index("c")
        base = wid * b_per_w

        pltpu.sync_copy(idx_hbm.at[pl.ds(base, b_per_w)], idx_v)

        def start_gather(b, g):
            pltpu.async_copy(
                table_hbm.at[idx_v.at[pl.ds(g * CHUNK, CHUNK)]], rows[b], gsem[b]
            )

        def wait_gather(b):
            pltpu.make_async_copy(
                out_hbm.at[pl.ds(base, CHUNK)], rows[b], gsem[b]
            ).wait()

        def scale_rows(b):
            @plsc.parallel_loop(0, CHUNK, unroll=8)
            def _(r):
                for k in range(D_MODEL // 16):
                    sl = pl.ds(k * 16, 16)
                    rows[b][r, sl] = rows[b][r, sl] * SCALE

        def start_store(b, g):
            pltpu.async_copy(rows[b], out_hbm.at[pl.ds(base + g * CHUNK, CHUNK)], osem[b])

        def wait_store(b):
            pltpu.make_async_copy(
                rows[b], out_hbm.at[pl.ds(base, CHUNK)], osem[b]
            ).wait()

        for b in range(NBUF):
            start_gather(b, b)

        def outer(p, carry):
            for b in range(NBUF):
                g = p * NBUF + b
                wait_gather(b)
                scale_rows(b)
                start_store(b, g)
                wait_store(b)
                start_gather(b, g + NBUF)
            return carry

        lax.fori_loop(0, n_outer - 1, outer, 0)

        for b in range(NBUF):
            g = (n_outer - 1) * NBUF + b
            wait_gather(b)
            scale_rows(b)
            start_store(b, g)
        for b in range(NBUF):
            wait_store(b)

    return emb(idx_flat, table)


def kernel(x, table):
    rows, cols = x.shape
    total = rows * cols
    idx_flat = x.reshape(total).astype(jnp.int32)
    out = _embed(idx_flat, table, total)
    return out.reshape(rows, cols, D_MODEL)
